# baseline (device time: 65569 ns/iter reference)
import jax
import jax.numpy as jnp
from jax import lax
from jax.experimental import pallas as pl
from jax.experimental.pallas import tpu as pltpu

import os

N_CHUNKS = 16
SCALE = 6.0 / 127.0
INV_SCALE = 127.0 / 6.0

_VARIANT = os.environ.get("KVARIANT", "full")


def kernel(x):
    m, n = x.shape
    half_n = n // 2
    rows = m // N_CHUNKS

    def body(
        x_ref,
        out_ref,
        x_vm,
        q_send,
        q_recv,
        local_buf,
        deq_buf,
        in_sems,
        out_sems,
        deq_out_sems,
        send_sems,
        recv_sems,
    ):
        xi = lax.axis_index("x")
        yi = lax.axis_index("y")
        zi = lax.axis_index("z")

        def in_dma(c):
            return pltpu.make_async_copy(
                x_ref.at[pl.ds(c * rows, rows), :],
                x_vm.at[c % 2],
                in_sems.at[c % 2],
            )

        def make_branch(my_z):
            other_z = 1 - my_z
            no_math = _VARIANT in ("nocompute", "oneway")
            i_send = not (_VARIANT == "oneway" and my_z == 1)
            i_recv = not (_VARIANT == "oneway" and my_z == 0)

            def _pure_wire(nsend):
                barrier_sem = pltpu.get_barrier_semaphore()
                pl.semaphore_signal(
                    barrier_sem,
                    inc=1,
                    device_id=(xi, yi, other_z),
                    device_id_type=pl.DeviceIdType.MESH,
                )
                pl.semaphore_wait(barrier_sem, 1)
                blk = m // nsend
                rdmas = [
                    pltpu.make_async_remote_copy(
                        src_ref=q_send.at[pl.ds(k * blk, blk), :],
                        dst_ref=q_recv.at[pl.ds(k * blk, blk), :],
                        send_sem=send_sems.at[k % N_CHUNKS],
                        recv_sem=recv_sems.at[k % N_CHUNKS],
                        device_id=(xi, yi, other_z),
                        device_id_type=pl.DeviceIdType.MESH,
                    )
                    for k in range(nsend)
                ]
                for r in rdmas:
                    r.start()
                for r in rdmas:
                    r.wait_recv()
                for r in rdmas:
                    r.wait_send()

            def _():
                if _VARIANT == "purewire1":
                    return _pure_wire(1)
                if _VARIANT == "purewire16":
                    return _pure_wire(16)
                in_dma(0).start()
                barrier_sem = pltpu.get_barrier_semaphore()
                pl.semaphore_signal(
                    barrier_sem,
                    inc=1,
                    device_id=(xi, yi, other_z),
                    device_id_type=pl.DeviceIdType.MESH,
                )
                pl.semaphore_wait(barrier_sem, 1)

                def send_chunk(c):
                    r0 = c * rows
                    if _VARIANT == "nowire":
                        pltpu.make_async_copy(
                            q_send.at[pl.ds(r0, rows), :],
                            q_recv.at[pl.ds(r0, rows), :],
                            recv_sems.at[c],
                        ).start()
                    else:
                        pltpu.make_async_remote_copy(
                            src_ref=q_send.at[pl.ds(r0, rows), :],
                            dst_ref=q_recv.at[pl.ds(r0, rows), :],
                            send_sem=send_sems.at[c],
                            recv_sem=recv_sems.at[c],
                            device_id=(xi, yi, other_z),
                            device_id_type=pl.DeviceIdType.MESH,
                        ).start()

                def wait_chunk_recv(c):
                    r0 = c * rows
                    if _VARIANT == "nowire":
                        pltpu.make_async_copy(
                            q_send.at[pl.ds(r0, rows), :],
                            q_recv.at[pl.ds(r0, rows), :],
                            recv_sems.at[c],
                        ).wait()
                    else:
                        pltpu.make_async_remote_copy(
                            src_ref=q_send.at[pl.ds(r0, rows), :],
                            dst_ref=q_recv.at[pl.ds(r0, rows), :],
                            send_sem=send_sems.at[c],
                            recv_sem=recv_sems.at[c],
                            device_id=(xi, yi, other_z),
                            device_id_type=pl.DeviceIdType.MESH,
                        ).wait_recv()

                def dequant(c):
                    r0 = c * rows
                    wait_chunk_recv(c)
                    if not no_math:
                        deq_buf[pl.ds(r0, rows), :] = (
                            q_recv[pl.ds(r0, rows), :].astype(jnp.bfloat16)
                            * SCALE
                        )
                    pltpu.make_async_copy(
                        deq_buf.at[pl.ds(r0, rows), :],
                        out_ref.at[pl.ds(other_z * m + r0, rows), :],
                        deq_out_sems.at[c],
                    ).start()

                DEQ_LAG = 3

                for c in range(N_CHUNKS):
                    r0 = c * rows
                    if c + 1 < N_CHUNKS:
                        in_dma(c + 1).start()
                    in_dma(c).wait()

                    if not no_math:
                        xc = x_vm[c % 2, :, pl.ds(other_z * half_n, half_n)]
                        q_send[pl.ds(r0, rows), :] = lax.round(
                            xc * INV_SCALE, lax.RoundingMethod.TO_NEAREST_EVEN
                        ).astype(jnp.int8)

                    if i_send:
                        send_chunk(c)

                    if not no_math:
                        local_buf[pl.ds(r0, rows), :] = x_vm[
                            c % 2, :, pl.ds(my_z * half_n, half_n)
                        ].astype(jnp.bfloat16)
                    pltpu.make_async_copy(
                        local_buf.at[pl.ds(r0, rows), :],
                        out_ref.at[pl.ds(my_z * m + r0, rows), :],
                        out_sems.at[c],
                    ).start()

                    if i_recv and c >= DEQ_LAG:
                        dequant(c - DEQ_LAG)

                if i_recv:
                    for c in range(N_CHUNKS - DEQ_LAG, N_CHUNKS):
                        dequant(c)

                for c in range(N_CHUNKS):
                    r0 = c * rows
                    if i_send and _VARIANT != "nowire":
                        pltpu.make_async_remote_copy(
                            src_ref=q_send.at[pl.ds(r0, rows), :],
                            dst_ref=q_recv.at[pl.ds(r0, rows), :],
                            send_sem=send_sems.at[c],
                            recv_sem=recv_sems.at[c],
                            device_id=(xi, yi, other_z),
                            device_id_type=pl.DeviceIdType.MESH,
                        ).wait_send()
                    pltpu.make_async_copy(
                        local_buf.at[pl.ds(r0, rows), :],
                        out_ref.at[pl.ds(my_z * m + r0, rows), :],
                        out_sems.at[c],
                    ).wait()
                    if i_recv:
                        pltpu.make_async_copy(
                            deq_buf.at[pl.ds(r0, rows), :],
                            out_ref.at[pl.ds(other_z * m + r0, rows), :],
                            deq_out_sems.at[c],
                        ).wait()

            return _

        pl.when(zi == 0)(make_branch(0))
        pl.when(zi == 1)(make_branch(1))

    return pl.pallas_call(
        body,
        out_shape=jax.ShapeDtypeStruct((2 * m, half_n), jnp.bfloat16),
        in_specs=[pl.BlockSpec(memory_space=pl.ANY)],
        out_specs=pl.BlockSpec(memory_space=pl.ANY),
        scratch_shapes=[
            pltpu.VMEM((2, rows, n), jnp.float32),
            pltpu.VMEM((m, half_n), jnp.int8),
            pltpu.VMEM((m, half_n), jnp.int8),
            pltpu.VMEM((m, half_n), jnp.bfloat16),
            pltpu.VMEM((m, half_n), jnp.bfloat16),
            pltpu.SemaphoreType.DMA((2,)),
            pltpu.SemaphoreType.DMA((N_CHUNKS,)),
            pltpu.SemaphoreType.DMA((N_CHUNKS,)),
            pltpu.SemaphoreType.DMA((N_CHUNKS,)),
            pltpu.SemaphoreType.DMA((N_CHUNKS,)),
        ],
        compiler_params=pltpu.CompilerParams(
            vmem_limit_bytes=100 * 1024 * 1024,
            collective_id=0,
        ),
    )(x)


# device time: 65380 ns/iter; 1.0029x vs baseline; 1.0029x over previous
import jax
import jax.numpy as jnp
from jax import lax
from jax.experimental import pallas as pl
from jax.experimental.pallas import tpu as pltpu

import os

N_CHUNKS = 16
SCALE = 6.0 / 127.0
INV_SCALE = 127.0 / 6.0

_VARIANT = os.environ.get("KVARIANT", "full")


def kernel(x):
    m, n = x.shape
    half_n = n // 2
    rows = m // N_CHUNKS

    HEAD = rows // 8
    segs = [(0, HEAD), (HEAD, rows - HEAD)]
    segs += [(c * rows, rows) for c in range(1, N_CHUNKS - 1)]
    segs += [((N_CHUNKS - 1) * rows, rows - HEAD), (m - HEAD, HEAD)]
    n_segs = len(segs)
    segs_of_chunk = [
        [s for s, (r0, _) in enumerate(segs) if r0 // rows == c]
        for c in range(N_CHUNKS)
    ]

    def body(
        x_ref,
        out_ref,
        x_vm,
        q_send,
        q_recv,
        local_buf,
        deq_buf,
        in_sems,
        out_sems,
        deq_out_sems,
        send_sems,
        recv_sems,
    ):
        xi = lax.axis_index("x")
        yi = lax.axis_index("y")
        zi = lax.axis_index("z")

        def in_dma(c):
            return pltpu.make_async_copy(
                x_ref.at[pl.ds(c * rows, rows), :],
                x_vm.at[c % 2],
                in_sems.at[c % 2],
            )

        def make_branch(my_z):
            other_z = 1 - my_z
            no_math = _VARIANT in ("nocompute", "oneway")
            i_send = not (_VARIANT == "oneway" and my_z == 1)
            i_recv = not (_VARIANT == "oneway" and my_z == 0)

            def _pure_wire(nsend):
                barrier_sem = pltpu.get_barrier_semaphore()
                pl.semaphore_signal(
                    barrier_sem,
                    inc=1,
                    device_id=(xi, yi, other_z),
                    device_id_type=pl.DeviceIdType.MESH,
                )
                pl.semaphore_wait(barrier_sem, 1)
                blk = m // nsend
                rdmas = [
                    pltpu.make_async_remote_copy(
                        src_ref=q_send.at[pl.ds(k * blk, blk), :],
                        dst_ref=q_recv.at[pl.ds(k * blk, blk), :],
                        send_sem=send_sems.at[k % N_CHUNKS],
                        recv_sem=recv_sems.at[k % N_CHUNKS],
                        device_id=(xi, yi, other_z),
                        device_id_type=pl.DeviceIdType.MESH,
                    )
                    for k in range(nsend)
                ]
                for r in rdmas:
                    r.start()
                for r in rdmas:
                    r.wait_recv()
                for r in rdmas:
                    r.wait_send()

            def _():
                if _VARIANT == "purewire1":
                    return _pure_wire(1)
                if _VARIANT == "purewire16":
                    return _pure_wire(16)
                in_dma(0).start()
                barrier_sem = pltpu.get_barrier_semaphore()
                pl.semaphore_signal(
                    barrier_sem,
                    inc=1,
                    device_id=(xi, yi, other_z),
                    device_id_type=pl.DeviceIdType.MESH,
                )
                pl.semaphore_wait(barrier_sem, 1)

                def seg_rdma(s):
                    r0, nr = segs[s]
                    return pltpu.make_async_remote_copy(
                        src_ref=q_send.at[pl.ds(r0, nr), :],
                        dst_ref=q_recv.at[pl.ds(r0, nr), :],
                        send_sem=send_sems.at[s],
                        recv_sem=recv_sems.at[s],
                        device_id=(xi, yi, other_z),
                        device_id_type=pl.DeviceIdType.MESH,
                    )

                def seg_local_copy(s):
                    r0, nr = segs[s]
                    return pltpu.make_async_copy(
                        q_send.at[pl.ds(r0, nr), :],
                        q_recv.at[pl.ds(r0, nr), :],
                        recv_sems.at[s],
                    )

                def send_seg(s):
                    if _VARIANT == "nowire":
                        seg_local_copy(s).start()
                    else:
                        seg_rdma(s).start()

                def dequant_seg(s):
                    r0, nr = segs[s]
                    if _VARIANT == "nowire":
                        seg_local_copy(s).wait()
                    else:
                        seg_rdma(s).wait_recv()
                    if not no_math:
                        deq_buf[pl.ds(r0, nr), :] = (
                            q_recv[pl.ds(r0, nr), :].astype(jnp.bfloat16)
                            * SCALE
                        )
                    pltpu.make_async_copy(
                        deq_buf.at[pl.ds(r0, nr), :],
                        out_ref.at[pl.ds(other_z * m + r0, nr), :],
                        deq_out_sems.at[s],
                    ).start()

                def quant_seg(c, s):
                    r0, nr = segs[s]
                    lr = r0 - c * rows
                    xc = x_vm[
                        c % 2, pl.ds(lr, nr), pl.ds(other_z * half_n, half_n)
                    ]
                    q_send[pl.ds(r0, nr), :] = lax.round(
                        xc * INV_SCALE, lax.RoundingMethod.TO_NEAREST_EVEN
                    ).astype(jnp.int8)

                DEQ_LAG_SEGS = 3
                sent = 0
                deq_cursor = 0

                for c in range(N_CHUNKS):
                    r0 = c * rows
                    if c + 1 < N_CHUNKS:
                        in_dma(c + 1).start()
                    in_dma(c).wait()

                    for s in segs_of_chunk[c]:
                        if not no_math:
                            quant_seg(c, s)
                        if i_send:
                            send_seg(s)
                        sent += 1

                    if not no_math:
                        local_buf[pl.ds(r0, rows), :] = x_vm[
                            c % 2, :, pl.ds(my_z * half_n, half_n)
                        ].astype(jnp.bfloat16)
                    pltpu.make_async_copy(
                        local_buf.at[pl.ds(r0, rows), :],
                        out_ref.at[pl.ds(my_z * m + r0, rows), :],
                        out_sems.at[c],
                    ).start()

                    if i_recv:
                        while deq_cursor < sent - DEQ_LAG_SEGS:
                            dequant_seg(deq_cursor)
                            deq_cursor += 1

                if i_recv:
                    while deq_cursor < n_segs:
                        dequant_seg(deq_cursor)
                        deq_cursor += 1

                if i_send and _VARIANT != "nowire":
                    for s in range(n_segs):
                        seg_rdma(s).wait_send()
                for c in range(N_CHUNKS):
                    r0 = c * rows
                    pltpu.make_async_copy(
                        local_buf.at[pl.ds(r0, rows), :],
                        out_ref.at[pl.ds(my_z * m + r0, rows), :],
                        out_sems.at[c],
                    ).wait()
                if i_recv:
                    for s in range(n_segs):
                        r0, nr = segs[s]
                        pltpu.make_async_copy(
                            deq_buf.at[pl.ds(r0, nr), :],
                            out_ref.at[pl.ds(other_z * m + r0, nr), :],
                            deq_out_sems.at[s],
                        ).wait()

            return _

        pl.when(zi == 0)(make_branch(0))
        pl.when(zi == 1)(make_branch(1))

    return pl.pallas_call(
        body,
        out_shape=jax.ShapeDtypeStruct((2 * m, half_n), jnp.bfloat16),
        in_specs=[pl.BlockSpec(memory_space=pl.ANY)],
        out_specs=pl.BlockSpec(memory_space=pl.ANY),
        scratch_shapes=[
            pltpu.VMEM((2, rows, n), jnp.float32),
            pltpu.VMEM((m, half_n), jnp.int8),
            pltpu.VMEM((m, half_n), jnp.int8),
            pltpu.VMEM((m, half_n), jnp.bfloat16),
            pltpu.VMEM((m, half_n), jnp.bfloat16),
            pltpu.SemaphoreType.DMA((2,)),
            pltpu.SemaphoreType.DMA((N_CHUNKS,)),
            pltpu.SemaphoreType.DMA((n_segs,)),
            pltpu.SemaphoreType.DMA((n_segs,)),
            pltpu.SemaphoreType.DMA((n_segs,)),
        ],
        compiler_params=pltpu.CompilerParams(
            vmem_limit_bytes=100 * 1024 * 1024,
            collective_id=0,
        ),
    )(x)
